# Initial kernel scaffold; baseline (speedup 1.0000x reference)
import jax
import jax.numpy as jnp
from jax.experimental import pallas as pl


def kernel(kpts, disp, pads, pads_values, simplices, coords, s):
    raise NotImplementedError("write your pallas kernel here")


# speedup vs baseline: 14.6252x; 14.6252x over previous
import functools

import jax
import jax.numpy as jnp
from jax import lax
from jax.experimental import pallas as pl
from jax.experimental.pallas import tpu as pltpu
from jax.experimental.pallas import tpu_sc as plsc

D, H, W = 96, 96, 96
M = D * H * W
N = 2048
T = 12288
NV = N + 8

NUM_TILES = 32
PER_TILE = M // NUM_TILES
NBLK = 8
BS = PER_TILE // NBLK
ITERS = BS // 16


def _interp_kernel(simp_hbm, vals_hbm, s_hbm, coords_hbm, out_hbm,
                   simp_v, vals_v, s_v, coords_v, out_v, sem):
    wid = lax.axis_index("s") * 2 + lax.axis_index("c")

    pltpu.sync_copy(simp_hbm, simp_v)
    pltpu.sync_copy(vals_hbm, vals_v)

    iot4 = lax.iota(jnp.int32, 16) * 4

    for b in range(NBLK):
        base = (wid * NBLK + b) * BS
        pltpu.sync_copy(s_hbm.at[pl.ds(base, BS)], s_v)
        pltpu.sync_copy(coords_hbm.at[pl.ds(base * 4, BS * 4)], coords_v)

        def body(i, carry):
            sv4 = s_v[pl.ds(i * 16, 16)] * 4
            cbase = iot4 + i * 64
            res0 = jnp.zeros((16,), jnp.float32)
            res1 = jnp.zeros((16,), jnp.float32)
            res2 = jnp.zeros((16,), jnp.float32)
            for j in range(4):
                vj = plsc.load_gather(simp_v, [sv4 + j])
                wj = plsc.load_gather(coords_v, [cbase + j])
                v3 = vj * 3
                res0 += wj * plsc.load_gather(vals_v, [v3])
                res1 += wj * plsc.load_gather(vals_v, [v3 + 1])
                res2 += wj * plsc.load_gather(vals_v, [v3 + 2])
            out_v[pl.ds(i * 16, 16)] = res0
            out_v[pl.ds(BS + i * 16, 16)] = res1
            out_v[pl.ds(2 * BS + i * 16, 16)] = res2
            return carry

        lax.fori_loop(0, ITERS, body, 0)

        for k in range(3):
            pltpu.sync_copy(out_v.at[pl.ds(k * BS, BS)],
                            out_hbm.at[pl.ds(k * M + base, BS)])


@jax.jit
def _run(simp_flat, vals_flat, s, coords_flat):
    mesh = plsc.VectorSubcoreMesh(core_axis_name="c", subcore_axis_name="s")
    kern = functools.partial(
        pl.kernel,
        out_type=jax.ShapeDtypeStruct((3 * M,), jnp.float32),
        mesh=mesh,
        compiler_params=pltpu.CompilerParams(needs_layout_passes=False),
        scratch_types=[
            pltpu.VMEM((T * 4,), jnp.int32),
            pltpu.VMEM((NV * 3,), jnp.float32),
            pltpu.VMEM((BS,), jnp.int32),
            pltpu.VMEM((BS * 4,), jnp.float32),
            pltpu.VMEM((3 * BS,), jnp.float32),
            pltpu.SemaphoreType.DMA,
        ],
    )(_interp_kernel)
    return kern(simp_flat, vals_flat, s, coords_flat)


def kernel(kpts, disp, pads, pads_values, simplices, coords, s):
    vals_flat = jnp.concatenate([disp[0], pads_values[0]], axis=0).reshape(-1)
    out_flat = _run(simplices.reshape(-1), vals_flat, s, coords.reshape(-1))
    return out_flat.reshape(1, 3, D, H, W)


# speedup vs baseline: 193.3634x; 13.2213x over previous
import functools

import jax
import jax.numpy as jnp
from jax import lax
from jax.experimental import pallas as pl
from jax.experimental.pallas import tpu as pltpu
from jax.experimental.pallas import tpu_sc as plsc

D, H, W = 96, 96, 96
M = D * H * W
N = 2048
T = 12288
NV = N + 8

NUM_TILES = 32
PER_TILE = M // NUM_TILES
NBLK = 8
BS = PER_TILE // NBLK
ITERS = BS // 16
ROWS = BS // W
OUT_BS = ROWS * 128
OUT_COMP = (M // W) * 128


def _interp_kernel(simp_hbm, vals_hbm, s_hbm, coords_hbm, out_hbm,
                   simp_v, vals_v, s_v, coords_v, out_v, sem):
    wid = lax.axis_index("s") * 2 + lax.axis_index("c")

    pltpu.sync_copy(simp_hbm, simp_v)
    pltpu.sync_copy(vals_hbm, vals_v)

    zv = jnp.zeros((16,), jnp.float32)
    for r in range(3 * ROWS):
        out_v[pl.ds(r * 128 + 96, 16)] = zv
        out_v[pl.ds(r * 128 + 112, 16)] = zv

    for b in range(NBLK):
        blk = wid * NBLK + b
        base = blk * BS
        pltpu.sync_copy(s_hbm.at[pl.ds(base, BS)], s_v)
        pltpu.sync_copy(coords_hbm.at[pl.ds(base * 4, BS * 4)], coords_v)

        def body(i, carry):
            sv = s_v[pl.ds(i * 16, 16)]
            sbase = ((sv >> 7) << 9) + (sv & 127)
            cslot = (i >> 3) * 512 + (i & 7) * 16
            res0 = jnp.zeros((16,), jnp.float32)
            res1 = jnp.zeros((16,), jnp.float32)
            res2 = jnp.zeros((16,), jnp.float32)
            for v in range(4):
                vert = plsc.load_gather(simp_v, [sbase + v * 128])
                wv = coords_v[pl.ds(cslot + v * 128, 16)]
                res0 += wv * plsc.load_gather(vals_v, [vert])
                res1 += wv * plsc.load_gather(vals_v, [vert + NV])
                res2 += wv * plsc.load_gather(vals_v, [vert + 2 * NV])
            obase = (i // 6) * 128 + (i % 6) * 16
            out_v[pl.ds(obase, 16)] = res0
            out_v[pl.ds(OUT_BS + obase, 16)] = res1
            out_v[pl.ds(2 * OUT_BS + obase, 16)] = res2
            return carry

        lax.fori_loop(0, ITERS, body, 0)

        row0 = blk * ROWS
        for k in range(3):
            pltpu.sync_copy(out_v.at[pl.ds(k * OUT_BS, OUT_BS)],
                            out_hbm.at[pl.ds(k * OUT_COMP + row0 * 128, OUT_BS)])


@jax.jit
def _run(simp_flat, vals_flat, s, coords_flat):
    mesh = plsc.VectorSubcoreMesh(core_axis_name="c", subcore_axis_name="s")
    kern = functools.partial(
        pl.kernel,
        out_type=jax.ShapeDtypeStruct((3 * OUT_COMP,), jnp.float32),
        mesh=mesh,
        compiler_params=pltpu.CompilerParams(needs_layout_passes=False),
        scratch_types=[
            pltpu.VMEM((T * 4,), jnp.int32),
            pltpu.VMEM((NV * 3,), jnp.float32),
            pltpu.VMEM((BS,), jnp.int32),
            pltpu.VMEM((BS * 4,), jnp.float32),
            pltpu.VMEM((3 * OUT_BS,), jnp.float32),
            pltpu.SemaphoreType.DMA,
        ],
    )(_interp_kernel)
    return kern(simp_flat, vals_flat, s, coords_flat)


def kernel(kpts, disp, pads, pads_values, simplices, coords, s):
    vals_flat = jnp.concatenate(
        [disp[0].T, pads_values[0].T], axis=1).reshape(-1)
    simp_flat = simplices.reshape(T // 128, 128, 4).transpose(0, 2, 1).reshape(-1)
    coords_flat = coords.reshape(M // 128, 128, 4).transpose(0, 2, 1).reshape(-1)
    out_flat = _run(simp_flat, vals_flat, s, coords_flat)
    return out_flat.reshape(1, 3, D, H, 128)[..., :W]


# speedup vs baseline: 311.8096x; 1.6126x over previous
import functools

import jax
import jax.numpy as jnp
from jax import lax
from jax.experimental import pallas as pl
from jax.experimental.pallas import tpu as pltpu
from jax.experimental.pallas import tpu_sc as plsc

D, H, W = 96, 96, 96
M = D * H * W
N = 2048
T = 12288
NV = N + 8

NUM_TILES = 32
PER_TILE = M // NUM_TILES
NBLK = 8
BS = PER_TILE // NBLK
ITERS = BS // 16
ROWS = BS // W
OUT_BS = ROWS * 128
OUT_COMP = (M // W) * 128


def _interp_kernel(simp_hbm, vals_hbm, s_hbm, coords_hbm, out_hbm,
                   simp_v, vals_v, s_v0, s_v1, coords_v0, coords_v1,
                   out_v0, out_v1, tbl_sem, in_sem0, in_sem1,
                   out_sem0, out_sem1):
    wid = lax.axis_index("s") * 2 + lax.axis_index("c")
    s_bufs = (s_v0, s_v1)
    c_bufs = (coords_v0, coords_v1)
    o_bufs = (out_v0, out_v1)
    in_sems = (in_sem0, in_sem1)
    out_sems = (out_sem0, out_sem1)

    th0 = pltpu.async_copy(simp_hbm, simp_v, tbl_sem)
    th1 = pltpu.async_copy(vals_hbm, vals_v, tbl_sem)

    def start_in(b, p):
        base = (wid * NBLK + b) * BS
        h0 = pltpu.async_copy(s_hbm.at[pl.ds(base, BS)], s_bufs[p], in_sems[p])
        h1 = pltpu.async_copy(coords_hbm.at[pl.ds(base * 4, BS * 4)],
                              c_bufs[p], in_sems[p])
        return (h0, h1)

    def start_out(b, p):
        row0 = (wid * NBLK + b) * ROWS
        return tuple(
            pltpu.async_copy(o_bufs[p].at[pl.ds(k * OUT_BS, OUT_BS)],
                             out_hbm.at[pl.ds(k * OUT_COMP + row0 * 128, OUT_BS)],
                             out_sems[p])
            for k in range(3))

    pend_in = [start_in(0, 0), start_in(1, 1)]
    pend_out = [None, None]

    zv = jnp.zeros((16,), jnp.float32)
    for o_v in o_bufs:
        for r in range(3 * ROWS):
            o_v[pl.ds(r * 128 + 96, 16)] = zv
            o_v[pl.ds(r * 128 + 112, 16)] = zv

    th0.wait()
    th1.wait()

    for b in range(NBLK):
        p = b & 1
        for h in pend_in[p]:
            h.wait()
        if pend_out[p] is not None:
            for h in pend_out[p]:
                h.wait()
        s_v, coords_v, out_v = s_bufs[p], c_bufs[p], o_bufs[p]

        @plsc.parallel_loop(0, ITERS, unroll=4)
        def body(i):
            sv = s_v[pl.ds(i * 16, 16)]
            sbase = ((sv >> 7) << 9) + (sv & 127)
            cslot = (i >> 3) * 512 + (i & 7) * 16
            res0 = jnp.zeros((16,), jnp.float32)
            res1 = jnp.zeros((16,), jnp.float32)
            res2 = jnp.zeros((16,), jnp.float32)
            for v in range(4):
                vert = plsc.load_gather(simp_v, [sbase + v * 128])
                wv = coords_v[pl.ds(cslot + v * 128, 16)]
                res0 += wv * plsc.load_gather(vals_v, [vert])
                res1 += wv * plsc.load_gather(vals_v, [vert + NV])
                res2 += wv * plsc.load_gather(vals_v, [vert + 2 * NV])
            obase = (i // 6) * 128 + (i % 6) * 16
            out_v[pl.ds(obase, 16)] = res0
            out_v[pl.ds(OUT_BS + obase, 16)] = res1
            out_v[pl.ds(2 * OUT_BS + obase, 16)] = res2

        if b + 2 < NBLK:
            pend_in[p] = start_in(b + 2, p)
        pend_out[p] = start_out(b, p)

    for h in pend_out[0]:
        h.wait()
    for h in pend_out[1]:
        h.wait()


@jax.jit
def _run(simp_flat, vals_flat, s, coords_flat):
    mesh = plsc.VectorSubcoreMesh(core_axis_name="c", subcore_axis_name="s")
    kern = functools.partial(
        pl.kernel,
        out_type=jax.ShapeDtypeStruct((3 * OUT_COMP,), jnp.float32),
        mesh=mesh,
        compiler_params=pltpu.CompilerParams(needs_layout_passes=False),
        scratch_types=[
            pltpu.VMEM((T * 4,), jnp.int32),
            pltpu.VMEM((NV * 3,), jnp.float32),
            pltpu.VMEM((BS,), jnp.int32),
            pltpu.VMEM((BS,), jnp.int32),
            pltpu.VMEM((BS * 4,), jnp.float32),
            pltpu.VMEM((BS * 4,), jnp.float32),
            pltpu.VMEM((3 * OUT_BS,), jnp.float32),
            pltpu.VMEM((3 * OUT_BS,), jnp.float32),
            pltpu.SemaphoreType.DMA,
            pltpu.SemaphoreType.DMA,
            pltpu.SemaphoreType.DMA,
            pltpu.SemaphoreType.DMA,
            pltpu.SemaphoreType.DMA,
        ],
    )(_interp_kernel)
    return kern(simp_flat, vals_flat, s, coords_flat)


def kernel(kpts, disp, pads, pads_values, simplices, coords, s):
    vals_flat = jnp.concatenate(
        [disp[0].T, pads_values[0].T], axis=1).reshape(-1)
    simp_flat = simplices.reshape(T // 128, 128, 4).transpose(0, 2, 1).reshape(-1)
    coords_flat = coords.reshape(M // 128, 128, 4).transpose(0, 2, 1).reshape(-1)
    out_flat = _run(simp_flat, vals_flat, s, coords_flat)
    return out_flat.reshape(1, 3, D, H, 128)[..., :W]
